# Initial kernel scaffold; baseline (speedup 1.0000x reference)
import jax
import jax.numpy as jnp
from jax.experimental import pallas as pl


def kernel(x, edge_index, W_emb, b_emb, Wg0, bg0, Wg1, bg1, Wg2, bg2, Wg3, bg3, Ws0, bs0, Ws1, bs1, Ws2, bs2, W_ro):
    raise NotImplementedError("write your pallas kernel here")


# speedup vs baseline: 2.3456x; 2.3456x over previous
import functools

import jax
import jax.numpy as jnp
from jax import lax
from jax.experimental import pallas as pl
from jax.experimental.pallas import tpu as pltpu
from jax.experimental.pallas import tpu_sc as plsc

N = 10000
E = 320000
D = 128
DH = 64

NC = 2
NS = 16
NW = NC * NS

K = 128
NP = 10240
EP = 327680
EPW = EP // NW
NCHUNK = EPW // K
EPT = EP // NS
NCHUNK2 = EPT // K
RPT = NP // NS

BM = 1024
GRID = NP // BM

_mesh = plsc.VectorSubcoreMesh(core_axis_name="c", subcore_axis_name="s")


def _make_segsum(table_rows, nchunk):
    @functools.partial(
        pl.kernel,
        out_type=jax.ShapeDtypeStruct((2 * NP, DH), jnp.float32),
        mesh=_mesh,
        compiler_params=pltpu.CompilerParams(use_tc_tiling_on_sc=False),
        scratch_types=[
            pltpu.VMEM((nchunk, K), jnp.int32),
            pltpu.VMEM((nchunk, K), jnp.int32),
            pltpu.VMEM((K, DH), jnp.float32),
            pltpu.VMEM_SHARED((NP, DH), jnp.float32),
            pltpu.SemaphoreType.DMA,
        ],
    )
    def segsum(m_hbm, src_hbm, dst_hbm, zero_hbm, out_hbm,
               src_v, dst_v, rows_v, acc, sem):
        cid = lax.axis_index("c")
        sid = lax.axis_index("s")
        pltpu.sync_copy(src_hbm.at[cid, sid], src_v)
        pltpu.sync_copy(dst_hbm.at[cid, sid], dst_v)
        pltpu.sync_copy(zero_hbm.at[pl.ds(sid * RPT, RPT)],
                        acc.at[pl.ds(sid * RPT, RPT)])
        plsc.subcore_barrier()

        def body(j, carry):
            pltpu.async_copy(m_hbm.at[src_v.at[j]], rows_v, sem).wait()
            pltpu.sync_copy(rows_v, acc.at[dst_v.at[j]], add=True)
            return carry

        lax.fori_loop(0, nchunk, body, 0)
        plsc.subcore_barrier()
        pltpu.sync_copy(acc.at[pl.ds(sid * RPT, RPT)],
                        out_hbm.at[pl.ds(cid * NP + sid * RPT, RPT)])

    return segsum


_seg_split = _make_segsum(NP, NCHUNK)
_seg_pair = _make_segsum(2 * NP, NCHUNK2)


def _dot(a, b):
    return jnp.dot(a, b, preferred_element_type=jnp.float32)


def _embed_body(x_ref, we_ref, be_ref, wg_ref, ma_ref, mb_ref):
    h0 = _dot(x_ref[...], we_ref[...]) + be_ref[...]
    m = _dot(h0, wg_ref[...])
    ma_ref[...] = m[:, :DH]
    mb_ref[...] = m[:, DH:]


_embed_mm = pl.pallas_call(
    _embed_body,
    grid=(GRID,),
    in_specs=[pl.BlockSpec((BM, D), lambda i: (i, 0)),
              pl.BlockSpec((D, D), lambda i: (0, 0)),
              pl.BlockSpec((1, D), lambda i: (0, 0)),
              pl.BlockSpec((D, D), lambda i: (0, 0))],
    out_specs=[pl.BlockSpec((BM, DH), lambda i: (i, 0)),
               pl.BlockSpec((BM, DH), lambda i: (i, 0))],
    out_shape=[jax.ShapeDtypeStruct((NP, DH), jnp.float32),
               jax.ShapeDtypeStruct((NP, DH), jnp.float32)],
)


def _post1_body(pa0_ref, pa1_ref, pb0_ref, pb1_ref, b_ref, w2_ref,
                ma_ref, mb_ref):
    aggA = pa0_ref[...] + pa1_ref[...]
    aggB = pb0_ref[...] + pb1_ref[...]
    h = jnp.maximum(jnp.concatenate([aggA, aggB], axis=1) + b_ref[...], 0.0)
    m = _dot(h, w2_ref[0])
    ma_ref[...] = m[:, :DH]
    mb_ref[...] = m[:, DH:]


_post1 = pl.pallas_call(
    _post1_body,
    grid=(GRID, 2),
    in_specs=[pl.BlockSpec((BM, DH), lambda i, j: (i, 0)),
              pl.BlockSpec((BM, DH), lambda i, j: (i + GRID, 0)),
              pl.BlockSpec((BM, DH), lambda i, j: (i, 0)),
              pl.BlockSpec((BM, DH), lambda i, j: (i + GRID, 0)),
              pl.BlockSpec((1, D), lambda i, j: (0, 0)),
              pl.BlockSpec((1, D, D), lambda i, j: (j, 0, 0))],
    out_specs=[pl.BlockSpec((BM, DH), lambda i, j: (i + j * GRID, 0)),
               pl.BlockSpec((BM, DH), lambda i, j: (i + j * GRID, 0))],
    out_shape=[jax.ShapeDtypeStruct((2 * NP, DH), jnp.float32),
               jax.ShapeDtypeStruct((2 * NP, DH), jnp.float32)],
)


def _post2_body(ga_ref, gb_ref, sa_ref, sb_ref, bg_ref, bs_ref, w2_ref,
                ma_ref, mb_ref, s_ref):
    g = jnp.concatenate([ga_ref[...], gb_ref[...]], axis=1)
    h = jnp.maximum(g + bg_ref[...], 0.0)
    m = _dot(h, w2_ref[0])
    ma_ref[...] = m[:, :DH]
    mb_ref[...] = m[:, DH:]
    sfull = jnp.concatenate([sa_ref[...], sb_ref[...]], axis=1)
    s_ref[...] = jnp.maximum(sfull + bs_ref[...], 0.0)


_post2 = pl.pallas_call(
    _post2_body,
    grid=(GRID, 2),
    in_specs=[pl.BlockSpec((BM, DH), lambda i, j: (i, 0)),
              pl.BlockSpec((BM, DH), lambda i, j: (i, 0)),
              pl.BlockSpec((BM, DH), lambda i, j: (i + GRID, 0)),
              pl.BlockSpec((BM, DH), lambda i, j: (i + GRID, 0)),
              pl.BlockSpec((1, D), lambda i, j: (0, 0)),
              pl.BlockSpec((1, D), lambda i, j: (0, 0)),
              pl.BlockSpec((1, D, D), lambda i, j: (j, 0, 0))],
    out_specs=[pl.BlockSpec((BM, DH), lambda i, j: (i + j * GRID, 0)),
               pl.BlockSpec((BM, DH), lambda i, j: (i + j * GRID, 0)),
               pl.BlockSpec((BM, D), lambda i, j: (i, 0))],
    out_shape=[jax.ShapeDtypeStruct((2 * NP, DH), jnp.float32),
               jax.ShapeDtypeStruct((2 * NP, DH), jnp.float32),
               jax.ShapeDtypeStruct((NP, D), jnp.float32)],
)


def _post3_body(ga_ref, gb_ref, sa_ref, sb_ref, bg_ref, bs_ref, w_ref,
                ma_ref, mb_ref, s_ref):
    g = jnp.concatenate([ga_ref[...], gb_ref[...]], axis=1)
    h = jnp.maximum(g + bg_ref[...], 0.0)
    m = _dot(h, w_ref[...])
    ma_ref[...] = m[:, :DH]
    mb_ref[...] = m[:, DH:]
    sfull = jnp.concatenate([sa_ref[...], sb_ref[...]], axis=1)
    s_ref[...] = jnp.maximum(sfull + bs_ref[...], 0.0)


_post3 = pl.pallas_call(
    _post3_body,
    grid=(GRID,),
    in_specs=[pl.BlockSpec((BM, DH), lambda i: (i, 0)),
              pl.BlockSpec((BM, DH), lambda i: (i, 0)),
              pl.BlockSpec((BM, DH), lambda i: (i + GRID, 0)),
              pl.BlockSpec((BM, DH), lambda i: (i + GRID, 0)),
              pl.BlockSpec((1, D), lambda i: (0, 0)),
              pl.BlockSpec((1, D), lambda i: (0, 0)),
              pl.BlockSpec((D, D), lambda i: (0, 0))],
    out_specs=[pl.BlockSpec((BM, DH), lambda i: (i, 0)),
               pl.BlockSpec((BM, DH), lambda i: (i, 0)),
               pl.BlockSpec((BM, D), lambda i: (i, 0))],
    out_shape=[jax.ShapeDtypeStruct((NP, DH), jnp.float32),
               jax.ShapeDtypeStruct((NP, DH), jnp.float32),
               jax.ShapeDtypeStruct((NP, D), jnp.float32)],
)


def _final_body(pa0_ref, pa1_ref, pb0_ref, pb1_ref, b_ref, s0_ref, s1_ref,
                wro_ref, s2_ref, lg_ref, sg_ref):
    aggA = pa0_ref[...] + pa1_ref[...]
    aggB = pb0_ref[...] + pb1_ref[...]
    s2 = jnp.maximum(jnp.concatenate([aggA, aggB], axis=1) + b_ref[...], 0.0)
    s2_ref[...] = s2
    of = jnp.concatenate([s0_ref[...], s1_ref[...], s2], axis=1)
    lg = _dot(of, wro_ref[...])
    lg_ref[...] = lg
    sg_ref[...] = jax.nn.sigmoid(lg)


_final = pl.pallas_call(
    _final_body,
    grid=(GRID,),
    in_specs=[pl.BlockSpec((BM, DH), lambda i: (i, 0)),
              pl.BlockSpec((BM, DH), lambda i: (i + GRID, 0)),
              pl.BlockSpec((BM, DH), lambda i: (i, 0)),
              pl.BlockSpec((BM, DH), lambda i: (i + GRID, 0)),
              pl.BlockSpec((1, D), lambda i: (0, 0)),
              pl.BlockSpec((BM, D), lambda i: (i, 0)),
              pl.BlockSpec((BM, D), lambda i: (i, 0)),
              pl.BlockSpec((3 * D, 1), lambda i: (0, 0))],
    out_specs=[pl.BlockSpec((BM, D), lambda i: (i, 0)),
               pl.BlockSpec((BM, 1), lambda i: (i, 0)),
               pl.BlockSpec((BM, 1), lambda i: (i, 0))],
    out_shape=[jax.ShapeDtypeStruct((NP, D), jnp.float32),
               jax.ShapeDtypeStruct((NP, 1), jnp.float32),
               jax.ShapeDtypeStruct((NP, 1), jnp.float32)],
)


def kernel(x, edge_index, W_emb, b_emb, Wg0, bg0, Wg1, bg1, Wg2, bg2,
           Wg3, bg3, Ws0, bs0, Ws1, bs1, Ws2, bs2, W_ro):
    ei = edge_index.astype(jnp.int32)
    padi = jnp.full((EP - E,), N, jnp.int32)
    srcp = jnp.concatenate([ei[0], padi])
    dstp = jnp.concatenate([ei[1], padi])
    src_sp = srcp.reshape(2, NS, NCHUNK, K)
    dst_sp = dstp.reshape(2, NS, NCHUNK, K)
    src_pr = jnp.stack([srcp, srcp + NP]).reshape(2, NS, NCHUNK2, K)
    dst_pr = jnp.stack([dstp, dstp]).reshape(2, NS, NCHUNK2, K)
    zeros = jnp.zeros((NP, DH), jnp.float32)
    xp = jnp.pad(x, ((0, NP - N), (0, 0)))

    be = b_emb.reshape(1, D)
    bg0r, bg1r, bg2r = bg0.reshape(1, D), bg1.reshape(1, D), bg2.reshape(1, D)
    bs0r, bs1r, bs2r = bs0.reshape(1, D), bs1.reshape(1, D), bs2.reshape(1, D)

    m0a, m0b = _embed_mm(xp, W_emb, be, Wg0)
    ppA = _seg_split(m0a, src_sp, dst_sp, zeros)
    ppB = _seg_split(m0b, src_sp, dst_sp, zeros)
    m1a, m1b = _post1(ppA, ppA, ppB, ppB, bg0r, jnp.stack([Wg1, Ws0]))
    aaA = _seg_pair(m1a, src_pr, dst_pr, zeros)
    aaB = _seg_pair(m1b, src_pr, dst_pr, zeros)
    m2a, m2b, s0 = _post2(aaA, aaB, aaA, aaB, bg1r, bs0r,
                          jnp.stack([Wg2, Ws1]))
    aaA2 = _seg_pair(m2a, src_pr, dst_pr, zeros)
    aaB2 = _seg_pair(m2b, src_pr, dst_pr, zeros)
    m3a, m3b, s1 = _post3(aaA2, aaB2, aaA2, aaB2, bg2r, bs1r, Ws2)
    pp3A = _seg_split(m3a, src_sp, dst_sp, zeros)
    pp3B = _seg_split(m3b, src_sp, dst_sp, zeros)
    s2, lg, sg = _final(pp3A, pp3A, pp3B, pp3B, bs2r, s0, s1, W_ro)

    out_feat = jnp.concatenate([s0[:N], s1[:N], s2[:N]], axis=1)
    return (out_feat, lg[:N, 0], sg[:N, 0])


# speedup vs baseline: 2.6351x; 1.1234x over previous
import functools

import jax
import jax.numpy as jnp
from jax import lax
from jax.experimental import pallas as pl
from jax.experimental.pallas import tpu as pltpu
from jax.experimental.pallas import tpu_sc as plsc

N = 10000
E = 320000
D = 128
DH = 64

NC = 2
NS = 16
NW = NC * NS

K = 128
NP = 10240
EP = 327680
EPW = EP // NW
NCHUNK = EPW // K
EPT = EP // NS
NCHUNK2 = EPT // K
RPT = NP // NS

BM = 1024
GRID = NP // BM

_mesh = plsc.VectorSubcoreMesh(core_axis_name="c", subcore_axis_name="s")


def _make_segsum(table_rows, nchunk):
    @functools.partial(
        pl.kernel,
        out_type=jax.ShapeDtypeStruct((2 * NP, DH), jnp.float32),
        mesh=_mesh,
        compiler_params=pltpu.CompilerParams(use_tc_tiling_on_sc=False),
        scratch_types=[
            pltpu.VMEM((nchunk, K), jnp.int32),
            pltpu.VMEM((nchunk, K), jnp.int32),
            pltpu.VMEM((K, DH), jnp.float32),
            pltpu.VMEM((K, DH), jnp.float32),
            pltpu.VMEM_SHARED((NP, DH), jnp.float32),
            pltpu.SemaphoreType.DMA,
            pltpu.SemaphoreType.DMA,
        ],
    )
    def segsum(m_hbm, src_hbm, dst_hbm, zero_hbm, out_hbm,
               src_v, dst_v, rows0, rows1, acc, sem0, sem1):
        cid = lax.axis_index("c")
        sid = lax.axis_index("s")
        pltpu.sync_copy(src_hbm.at[cid, sid], src_v)
        pltpu.sync_copy(dst_hbm.at[cid, sid], dst_v)
        pltpu.sync_copy(zero_hbm.at[pl.ds(sid * RPT, RPT)],
                        acc.at[pl.ds(sid * RPT, RPT)])
        plsc.subcore_barrier()

        pltpu.async_copy(m_hbm.at[src_v.at[0]], rows0, sem0)

        def body(t, carry):
            j0 = 2 * t
            j1 = j0 + 1
            pltpu.make_async_copy(m_hbm.at[src_v.at[j0]], rows0, sem0).wait()
            pltpu.async_copy(m_hbm.at[src_v.at[j1]], rows1, sem1)
            pltpu.sync_copy(rows0, acc.at[dst_v.at[j0]], add=True)
            pltpu.make_async_copy(m_hbm.at[src_v.at[j1]], rows1, sem1).wait()

            @pl.when(t < nchunk // 2 - 1)
            def _():
                pltpu.async_copy(m_hbm.at[src_v.at[j1 + 1]], rows0, sem0)

            pltpu.sync_copy(rows1, acc.at[dst_v.at[j1]], add=True)
            return carry

        lax.fori_loop(0, nchunk // 2, body, 0)
        plsc.subcore_barrier()
        pltpu.sync_copy(acc.at[pl.ds(sid * RPT, RPT)],
                        out_hbm.at[pl.ds(cid * NP + sid * RPT, RPT)])

    return segsum


_seg_split = _make_segsum(NP, NCHUNK)
_seg_pair = _make_segsum(2 * NP, NCHUNK2)


def _dot(a, b):
    return jnp.dot(a, b, preferred_element_type=jnp.float32)


def _embed_body(x_ref, we_ref, be_ref, wg_ref, ma_ref, mb_ref):
    h0 = _dot(x_ref[...], we_ref[...]) + be_ref[...]
    m = _dot(h0, wg_ref[...])
    ma_ref[...] = m[:, :DH]
    mb_ref[...] = m[:, DH:]


_embed_mm = pl.pallas_call(
    _embed_body,
    grid=(GRID,),
    in_specs=[pl.BlockSpec((BM, D), lambda i: (i, 0)),
              pl.BlockSpec((D, D), lambda i: (0, 0)),
              pl.BlockSpec((1, D), lambda i: (0, 0)),
              pl.BlockSpec((D, D), lambda i: (0, 0))],
    out_specs=[pl.BlockSpec((BM, DH), lambda i: (i, 0)),
               pl.BlockSpec((BM, DH), lambda i: (i, 0))],
    out_shape=[jax.ShapeDtypeStruct((NP, DH), jnp.float32),
               jax.ShapeDtypeStruct((NP, DH), jnp.float32)],
)


def _post1_body(pa0_ref, pa1_ref, pb0_ref, pb1_ref, b_ref, w2_ref,
                ma_ref, mb_ref):
    aggA = pa0_ref[...] + pa1_ref[...]
    aggB = pb0_ref[...] + pb1_ref[...]
    h = jnp.maximum(jnp.concatenate([aggA, aggB], axis=1) + b_ref[...], 0.0)
    m = _dot(h, w2_ref[0])
    ma_ref[...] = m[:, :DH]
    mb_ref[...] = m[:, DH:]


_post1 = pl.pallas_call(
    _post1_body,
    grid=(GRID, 2),
    in_specs=[pl.BlockSpec((BM, DH), lambda i, j: (i, 0)),
              pl.BlockSpec((BM, DH), lambda i, j: (i + GRID, 0)),
              pl.BlockSpec((BM, DH), lambda i, j: (i, 0)),
              pl.BlockSpec((BM, DH), lambda i, j: (i + GRID, 0)),
              pl.BlockSpec((1, D), lambda i, j: (0, 0)),
              pl.BlockSpec((1, D, D), lambda i, j: (j, 0, 0))],
    out_specs=[pl.BlockSpec((BM, DH), lambda i, j: (i + j * GRID, 0)),
               pl.BlockSpec((BM, DH), lambda i, j: (i + j * GRID, 0))],
    out_shape=[jax.ShapeDtypeStruct((2 * NP, DH), jnp.float32),
               jax.ShapeDtypeStruct((2 * NP, DH), jnp.float32)],
)


def _post2_body(ga_ref, gb_ref, sa_ref, sb_ref, bg_ref, bs_ref, w2_ref,
                ma_ref, mb_ref, s_ref):
    g = jnp.concatenate([ga_ref[...], gb_ref[...]], axis=1)
    h = jnp.maximum(g + bg_ref[...], 0.0)
    m = _dot(h, w2_ref[0])
    ma_ref[...] = m[:, :DH]
    mb_ref[...] = m[:, DH:]
    sfull = jnp.concatenate([sa_ref[...], sb_ref[...]], axis=1)
    s_ref[...] = jnp.maximum(sfull + bs_ref[...], 0.0)


_post2 = pl.pallas_call(
    _post2_body,
    grid=(GRID, 2),
    in_specs=[pl.BlockSpec((BM, DH), lambda i, j: (i, 0)),
              pl.BlockSpec((BM, DH), lambda i, j: (i, 0)),
              pl.BlockSpec((BM, DH), lambda i, j: (i + GRID, 0)),
              pl.BlockSpec((BM, DH), lambda i, j: (i + GRID, 0)),
              pl.BlockSpec((1, D), lambda i, j: (0, 0)),
              pl.BlockSpec((1, D), lambda i, j: (0, 0)),
              pl.BlockSpec((1, D, D), lambda i, j: (j, 0, 0))],
    out_specs=[pl.BlockSpec((BM, DH), lambda i, j: (i + j * GRID, 0)),
               pl.BlockSpec((BM, DH), lambda i, j: (i + j * GRID, 0)),
               pl.BlockSpec((BM, D), lambda i, j: (i, 0))],
    out_shape=[jax.ShapeDtypeStruct((2 * NP, DH), jnp.float32),
               jax.ShapeDtypeStruct((2 * NP, DH), jnp.float32),
               jax.ShapeDtypeStruct((NP, D), jnp.float32)],
)


def _post3_body(ga_ref, gb_ref, sa_ref, sb_ref, bg_ref, bs_ref, w_ref,
                ma_ref, mb_ref, s_ref):
    g = jnp.concatenate([ga_ref[...], gb_ref[...]], axis=1)
    h = jnp.maximum(g + bg_ref[...], 0.0)
    m = _dot(h, w_ref[...])
    ma_ref[...] = m[:, :DH]
    mb_ref[...] = m[:, DH:]
    sfull = jnp.concatenate([sa_ref[...], sb_ref[...]], axis=1)
    s_ref[...] = jnp.maximum(sfull + bs_ref[...], 0.0)


_post3 = pl.pallas_call(
    _post3_body,
    grid=(GRID,),
    in_specs=[pl.BlockSpec((BM, DH), lambda i: (i, 0)),
              pl.BlockSpec((BM, DH), lambda i: (i, 0)),
              pl.BlockSpec((BM, DH), lambda i: (i + GRID, 0)),
              pl.BlockSpec((BM, DH), lambda i: (i + GRID, 0)),
              pl.BlockSpec((1, D), lambda i: (0, 0)),
              pl.BlockSpec((1, D), lambda i: (0, 0)),
              pl.BlockSpec((D, D), lambda i: (0, 0))],
    out_specs=[pl.BlockSpec((BM, DH), lambda i: (i, 0)),
               pl.BlockSpec((BM, DH), lambda i: (i, 0)),
               pl.BlockSpec((BM, D), lambda i: (i, 0))],
    out_shape=[jax.ShapeDtypeStruct((NP, DH), jnp.float32),
               jax.ShapeDtypeStruct((NP, DH), jnp.float32),
               jax.ShapeDtypeStruct((NP, D), jnp.float32)],
)


def _final_body(pa0_ref, pa1_ref, pb0_ref, pb1_ref, b_ref, s0_ref, s1_ref,
                wro_ref, s2_ref, lg_ref, sg_ref):
    aggA = pa0_ref[...] + pa1_ref[...]
    aggB = pb0_ref[...] + pb1_ref[...]
    s2 = jnp.maximum(jnp.concatenate([aggA, aggB], axis=1) + b_ref[...], 0.0)
    s2_ref[...] = s2
    of = jnp.concatenate([s0_ref[...], s1_ref[...], s2], axis=1)
    lg = _dot(of, wro_ref[...])
    lg_ref[...] = lg
    sg_ref[...] = jax.nn.sigmoid(lg)


_final = pl.pallas_call(
    _final_body,
    grid=(GRID,),
    in_specs=[pl.BlockSpec((BM, DH), lambda i: (i, 0)),
              pl.BlockSpec((BM, DH), lambda i: (i + GRID, 0)),
              pl.BlockSpec((BM, DH), lambda i: (i, 0)),
              pl.BlockSpec((BM, DH), lambda i: (i + GRID, 0)),
              pl.BlockSpec((1, D), lambda i: (0, 0)),
              pl.BlockSpec((BM, D), lambda i: (i, 0)),
              pl.BlockSpec((BM, D), lambda i: (i, 0)),
              pl.BlockSpec((3 * D, 1), lambda i: (0, 0))],
    out_specs=[pl.BlockSpec((BM, D), lambda i: (i, 0)),
               pl.BlockSpec((BM, 1), lambda i: (i, 0)),
               pl.BlockSpec((BM, 1), lambda i: (i, 0))],
    out_shape=[jax.ShapeDtypeStruct((NP, D), jnp.float32),
               jax.ShapeDtypeStruct((NP, 1), jnp.float32),
               jax.ShapeDtypeStruct((NP, 1), jnp.float32)],
)


def kernel(x, edge_index, W_emb, b_emb, Wg0, bg0, Wg1, bg1, Wg2, bg2,
           Wg3, bg3, Ws0, bs0, Ws1, bs1, Ws2, bs2, W_ro):
    ei = edge_index.astype(jnp.int32)
    padi = jnp.full((EP - E,), N, jnp.int32)
    srcp = jnp.concatenate([ei[0], padi])
    dstp = jnp.concatenate([ei[1], padi])
    src_sp = srcp.reshape(2, NS, NCHUNK, K)
    dst_sp = dstp.reshape(2, NS, NCHUNK, K)
    src_pr = jnp.stack([srcp, srcp + NP]).reshape(2, NS, NCHUNK2, K)
    dst_pr = jnp.stack([dstp, dstp]).reshape(2, NS, NCHUNK2, K)
    zeros = jnp.zeros((NP, DH), jnp.float32)
    xp = jnp.pad(x, ((0, NP - N), (0, 0)))

    be = b_emb.reshape(1, D)
    bg0r, bg1r, bg2r = bg0.reshape(1, D), bg1.reshape(1, D), bg2.reshape(1, D)
    bs0r, bs1r, bs2r = bs0.reshape(1, D), bs1.reshape(1, D), bs2.reshape(1, D)

    m0a, m0b = _embed_mm(xp, W_emb, be, Wg0)
    ppA = _seg_split(m0a, src_sp, dst_sp, zeros)
    ppB = _seg_split(m0b, src_sp, dst_sp, zeros)
    m1a, m1b = _post1(ppA, ppA, ppB, ppB, bg0r, jnp.stack([Wg1, Ws0]))
    aaA = _seg_pair(m1a, src_pr, dst_pr, zeros)
    aaB = _seg_pair(m1b, src_pr, dst_pr, zeros)
    m2a, m2b, s0 = _post2(aaA, aaB, aaA, aaB, bg1r, bs0r,
                          jnp.stack([Wg2, Ws1]))
    aaA2 = _seg_pair(m2a, src_pr, dst_pr, zeros)
    aaB2 = _seg_pair(m2b, src_pr, dst_pr, zeros)
    m3a, m3b, s1 = _post3(aaA2, aaB2, aaA2, aaB2, bg2r, bs1r, Ws2)
    pp3A = _seg_split(m3a, src_sp, dst_sp, zeros)
    pp3B = _seg_split(m3b, src_sp, dst_sp, zeros)
    s2, lg, sg = _final(pp3A, pp3A, pp3B, pp3B, bs2r, s0, s1, W_ro)

    out_feat = jnp.concatenate([s0[:N], s1[:N], s2[:N]], axis=1)
    return (out_feat, lg[:N, 0], sg[:N, 0])
